# baseline (speedup 1.0000x reference)
import jax
import jax.numpy as jnp
from jax import lax
from jax.experimental import pallas as pl
from jax.experimental.pallas import tpu as pltpu
from jax.experimental.pallas import tpu_sc as plsc

B = 4096
FIELD = 26
MAXF = 20
EMB = 32
S = B * FIELD
NC, NS = 2, 16
NW = NC * NS
SEGS_PER_W = S // NW
CHUNK = 64
NCHUNK = SEGS_PER_W // CHUNK
IDS_PER_CHUNK = CHUNK * MAXF
IDX_ROWS = IDS_PER_CHUNK // 128
GATHER_ROWS = 64


ROWS_BYTES = IDS_PER_CHUNK * EMB * 4


def _sc_body(table_hbm, ids_hbm, lens_hbm, out_hbm,
             idx0, idx1, rows0, rows1, lens0, lens1, recip0, recip1,
             out_v, sem0, sem1):
    wid = lax.axis_index("s") * NC + lax.axis_index("c")
    seg0 = wid * SEGS_PER_W

    idx_b = (idx0, idx1)
    rows_b = (rows0, rows1)
    lens_b = (lens0, lens1)
    recip_b = (recip0, recip1)
    sem_b = (sem0, sem1)

    def prime(c, b):
        seg_base = seg0 + c * CHUNK
        pltpu.sync_copy(ids_hbm.at[pl.ds(seg_base * MAXF, IDS_PER_CHUNK)],
                        idx_b[b])
        for j in range(IDS_PER_CHUNK // GATHER_ROWS):
            pltpu.async_copy(
                table_hbm.at[idx_b[b].at[pl.ds(j * GATHER_ROWS, GATHER_ROWS)]],
                rows_b[b].at[pl.ds(j * GATHER_ROWS, GATHER_ROWS)],
                sem_b[b])
        pltpu.sync_copy(lens_hbm.at[pl.ds(seg_base, CHUNK)], lens_b[b])
        for g in range(CHUNK // 16):
            lv = lens_b[b][pl.ds(g * 16, 16)].astype(jnp.float32)
            recip_b[b][pl.ds(g * 16, 16)] = 1.0 / lv

    def consume(c, b):
        seg_base = seg0 + c * CHUNK
        pltpu.make_async_copy(
            out_hbm.at[pl.ds(0, IDS_PER_CHUNK)], rows_b[b], sem_b[b]).wait()

        rows_v = rows_b[b]
        recip_v = recip_b[b]

        @pl.loop(0, CHUNK, unroll=4)
        def seg_body(s):
            r0 = s * MAXF
            vals = [(rows_v[r0 + j, pl.ds(0, 16)],
                     rows_v[r0 + j, pl.ds(16, 16)]) for j in range(MAXF)]
            while len(vals) > 1:
                nxt = []
                for k in range(0, len(vals) - 1, 2):
                    nxt.append((vals[k][0] + vals[k + 1][0],
                                vals[k][1] + vals[k + 1][1]))
                if len(vals) % 2:
                    nxt.append(vals[-1])
                vals = nxt
            acc0, acc1 = vals[0]
            scale = plsc.load_gather(recip_v, [jnp.full((16,), s, jnp.int32)])
            out_v[s, pl.ds(0, 16)] = acc0 * scale
            out_v[s, pl.ds(16, 16)] = acc1 * scale
        pltpu.sync_copy(out_v, out_hbm.at[pl.ds(seg_base, CHUNK)])

    prime(0, 0)

    @pl.loop(0, NCHUNK, step=2)
    def chunk_loop(c2):
        for b in range(2):
            c = c2 + b
            prime(jnp.minimum(c + 1, NCHUNK - 1), 1 - b)
            consume(c, b)

    pltpu.make_async_copy(
        out_hbm.at[pl.ds(0, IDS_PER_CHUNK)], rows0, sem0).wait()


@jax.jit
def _dyemb(table, ids2d, lens):
    mesh = plsc.VectorSubcoreMesh(core_axis_name="c", subcore_axis_name="s",
                                  num_cores=NC, num_subcores=NS)
    return pl.kernel(
        _sc_body,
        out_type=jax.ShapeDtypeStruct((S, EMB), jnp.float32),
        mesh=mesh,
        scratch_types=[
            pltpu.VMEM((IDS_PER_CHUNK,), jnp.int32),
            pltpu.VMEM((IDS_PER_CHUNK,), jnp.int32),
            pltpu.VMEM((IDS_PER_CHUNK, EMB), jnp.float32),
            pltpu.VMEM((IDS_PER_CHUNK, EMB), jnp.float32),
            pltpu.VMEM((CHUNK,), jnp.int32),
            pltpu.VMEM((CHUNK,), jnp.int32),
            pltpu.VMEM((CHUNK,), jnp.float32),
            pltpu.VMEM((CHUNK,), jnp.float32),
            pltpu.VMEM((CHUNK, EMB), jnp.float32),
            pltpu.SemaphoreType.DMA,
            pltpu.SemaphoreType.DMA,
        ],
        compiler_params=pltpu.CompilerParams(needs_layout_passes=False,
                                             use_tc_tiling_on_sc=False),
    )(table, ids2d, lens)


def kernel(dynamic_ids, dynamic_lengths, embedding_weight):
    ids_flat = dynamic_ids.reshape(S * MAXF)
    lens = dynamic_lengths.reshape(S)
    out = _dyemb(embedding_weight, ids_flat, lens)
    return out.reshape(B, FIELD, EMB)


# speedup vs baseline: 1.0823x; 1.0823x over previous
import jax
import jax.numpy as jnp
from jax import lax
from jax.experimental import pallas as pl
from jax.experimental.pallas import tpu as pltpu
from jax.experimental.pallas import tpu_sc as plsc

B = 4096
FIELD = 26
MAXF = 20
EMB = 32
S = B * FIELD
NC, NS = 2, 16
NW = NC * NS
SEGS_PER_W = S // NW
CHUNK = 64
NCHUNK = SEGS_PER_W // CHUNK
IDS_PER_CHUNK = CHUNK * MAXF
IDX_ROWS = IDS_PER_CHUNK // 128
GATHER_ROWS = 64


ROWS_BYTES = IDS_PER_CHUNK * EMB * 4


def _sc_body(table_hbm, ids_hbm, lens_hbm, out_hbm,
             idx0, idx1, rows0, rows1, lens0, lens1, recip0, recip1,
             out_v, sem0, sem1, semi0, semi1):
    wid = lax.axis_index("s") * NC + lax.axis_index("c")
    seg0 = wid * SEGS_PER_W

    idx_b = (idx0, idx1)
    rows_b = (rows0, rows1)
    lens_b = (lens0, lens1)
    recip_b = (recip0, recip1)
    sem_b = (sem0, sem1)
    semi_b = (semi0, semi1)

    def ids_start(c, b):
        seg_base = seg0 + c * CHUNK
        pltpu.async_copy(ids_hbm.at[pl.ds(seg_base * MAXF, IDS_PER_CHUNK)],
                         idx_b[b], semi_b[b])

    def ids_drain(b):
        pltpu.make_async_copy(
            ids_hbm.at[pl.ds(0, IDS_PER_CHUNK)], idx_b[b], semi_b[b]).wait()

    def fire_gathers(b):
        for j in range(IDX_ROWS):
            pltpu.async_copy(
                table_hbm.at[idx_b[b].at[pl.ds(j * 128, 128)]],
                rows_b[b].at[pl.ds(j * 128, 128)],
                sem_b[b])

    def rows_drain(b):
        pltpu.make_async_copy(
            out_hbm.at[pl.ds(0, IDS_PER_CHUNK)], rows_b[b], sem_b[b]).wait()

    def stage_lens(c, b):
        seg_base = seg0 + c * CHUNK
        pltpu.sync_copy(lens_hbm.at[pl.ds(seg_base, CHUNK)], lens_b[b])
        for g in range(CHUNK // 16):
            lv = lens_b[b][pl.ds(g * 16, 16)].astype(jnp.float32)
            recip_b[b][pl.ds(g * 16, 16)] = 1.0 / lv

    def compute(c, b):
        seg_base = seg0 + c * CHUNK
        rows_v = rows_b[b]
        recip_v = recip_b[b]

        def seg_body(s, _):
            r0 = s * MAXF
            acc0 = rows_v[r0, pl.ds(0, 16)]
            acc1 = rows_v[r0, pl.ds(16, 16)]
            for j in range(1, MAXF):
                acc0 = acc0 + rows_v[r0 + j, pl.ds(0, 16)]
                acc1 = acc1 + rows_v[r0 + j, pl.ds(16, 16)]
            scale = plsc.load_gather(recip_v, [jnp.full((16,), s, jnp.int32)])
            out_v[s, pl.ds(0, 16)] = acc0 * scale
            out_v[s, pl.ds(16, 16)] = acc1 * scale
            return None

        lax.fori_loop(0, CHUNK, seg_body, None)
        pltpu.sync_copy(out_v, out_hbm.at[pl.ds(seg_base, CHUNK)])

    ids_start(0, 0)
    ids_drain(0)
    fire_gathers(0)
    ids_start(1, 1)
    stage_lens(0, 0)

    @pl.loop(0, NCHUNK, step=2)
    def chunk_loop(c2):
        for b in range(2):
            c = c2 + b
            nb = 1 - b
            c1 = jnp.minimum(c + 1, NCHUNK - 1)
            c2n = jnp.minimum(c + 2, NCHUNK - 1)
            ids_drain(nb)
            fire_gathers(nb)
            rows_drain(b)
            ids_start(c2n, b)
            stage_lens(c1, nb)
            compute(c, b)

    rows_drain(0)
    ids_drain(1)


@jax.jit
def _dyemb(table, ids2d, lens):
    mesh = plsc.VectorSubcoreMesh(core_axis_name="c", subcore_axis_name="s",
                                  num_cores=NC, num_subcores=NS)
    return pl.kernel(
        _sc_body,
        out_type=jax.ShapeDtypeStruct((S, EMB), jnp.float32),
        mesh=mesh,
        scratch_types=[
            pltpu.VMEM((IDS_PER_CHUNK,), jnp.int32),
            pltpu.VMEM((IDS_PER_CHUNK,), jnp.int32),
            pltpu.VMEM((IDS_PER_CHUNK, EMB), jnp.float32),
            pltpu.VMEM((IDS_PER_CHUNK, EMB), jnp.float32),
            pltpu.VMEM((CHUNK,), jnp.int32),
            pltpu.VMEM((CHUNK,), jnp.int32),
            pltpu.VMEM((CHUNK,), jnp.float32),
            pltpu.VMEM((CHUNK,), jnp.float32),
            pltpu.VMEM((CHUNK, EMB), jnp.float32),
            pltpu.SemaphoreType.DMA,
            pltpu.SemaphoreType.DMA,
            pltpu.SemaphoreType.DMA,
            pltpu.SemaphoreType.DMA,
        ],
        compiler_params=pltpu.CompilerParams(needs_layout_passes=False,
                                             use_tc_tiling_on_sc=False),
    )(table, ids2d, lens)


def kernel(dynamic_ids, dynamic_lengths, embedding_weight):
    ids_flat = dynamic_ids.reshape(S * MAXF)
    lens = dynamic_lengths.reshape(S)
    out = _dyemb(embedding_weight, ids_flat, lens)
    return out.reshape(B, FIELD, EMB)
